# baseline (speedup 1.0000x reference)
import functools

import jax
import jax.numpy as jnp
from jax import lax
from jax.experimental import pallas as pl
from jax.experimental.pallas import tpu as pltpu
from jax.experimental.pallas import tpu_sc as plsc

_CLOSE_IDX = 1


def _perf_body(x_ref, out_ref, *, lf, f, tt):
    x = x_ref[0]
    li = lax.broadcasted_iota(jnp.int32, (1, lf), 1)
    is_close = (li % f) == _CLOSE_IDX
    nz = is_close & (x != 0.0)
    lane_or_big = jnp.where(nz, li, lf + 1)
    minlane = jnp.min(lane_or_big, axis=1, keepdims=True)
    start = jnp.sum(jnp.where(li == minlane, x, 0.0), axis=1)
    last = jnp.sum(jnp.where(li == (lf - f + _CLOSE_IDX), x, 0.0), axis=1)
    safe = jnp.where(start != 0.0, start, 1.0)
    perf = jnp.where(start != 0.0, (last - start) / safe, 0.0)
    out_ref[0, 0, 0, :] = perf


def _compute_perf(ocr):
    b, t, lf = ocr.shape
    f = 5
    tt = 128
    nt = t // tt
    out = pl.pallas_call(
        functools.partial(_perf_body, lf=lf, f=f, tt=tt),
        grid=(b, nt),
        in_specs=[pl.BlockSpec((1, tt, lf), lambda i, j: (i, j, 0))],
        out_specs=pl.BlockSpec((1, 1, 1, tt), lambda i, j: (i, j, 0, 0)),
        out_shape=jax.ShapeDtypeStruct((b, nt, 1, tt), jnp.float32),
    )(ocr)
    return out.reshape(b, t)


def _rank_body(pfull_ref, pblk_ref, rank_ref, *, t, c):
    i0 = pl.program_id(1) * c
    pj = pfull_ref[...][:, None, :]
    pi = pblk_ref[...][:, :, None]
    jio = lax.broadcasted_iota(jnp.int32, (1, 1, t), 2)
    iio = lax.broadcasted_iota(jnp.int32, (1, c, 1), 1) + i0
    cmp = (pj > pi) | ((pj == pi) & (jio < iio))
    rank_ref[...] = jnp.sum(cmp.astype(jnp.int32), axis=2)


def _compute_rank(perf):
    b, t = perf.shape
    c = 128
    return pl.pallas_call(
        functools.partial(_rank_body, t=t, c=c),
        grid=(1, t // c),
        in_specs=[
            pl.BlockSpec((b, t), lambda z, i: (0, 0)),
            pl.BlockSpec((b, c), lambda z, i: (0, i)),
        ],
        out_specs=pl.BlockSpec((b, c), lambda z, i: (0, i)),
        out_shape=jax.ShapeDtypeStruct((b, t), jnp.int32),
    )(perf, perf)


def _orders_body(rank_ref, orders_ref, gidx_ref, *, t, c):
    k0 = pl.program_id(1) * c
    r = rank_ref[...][:, None, :]
    kio = lax.broadcasted_iota(jnp.int32, (1, c, 1), 1) + k0
    iio = lax.broadcasted_iota(jnp.int32, (1, 1, t), 2)
    orders = jnp.sum(jnp.where(r == kio, iio, 0), axis=2)
    orders_ref[...] = orders
    bio = lax.broadcasted_iota(jnp.int32, (orders.shape[0], 1), 0)
    gidx_ref[...] = orders + bio * t


def _compute_orders(rank):
    b, t = rank.shape
    c = 128
    return pl.pallas_call(
        functools.partial(_orders_body, t=t, c=c),
        grid=(1, t // c),
        in_specs=[pl.BlockSpec((b, t), lambda z, i: (0, 0))],
        out_specs=[
            pl.BlockSpec((b, c), lambda z, i: (0, i)),
            pl.BlockSpec((b, c), lambda z, i: (0, i)),
        ],
        out_shape=[
            jax.ShapeDtypeStruct((b, t), jnp.int32),
            jax.ShapeDtypeStruct((b, t), jnp.int32),
        ],
    )(rank)


def _make_sc_gather(n_rows, d01, d2):
    info = plsc.get_sparse_core_info()
    nc, ns = info.num_cores, info.num_subcores
    nw = nc * ns
    rpw = n_rows // nw
    chunk = 16
    nch = rpw // chunk

    mesh = plsc.VectorSubcoreMesh(core_axis_name="c", subcore_axis_name="s")

    @functools.partial(
        pl.kernel,
        mesh=mesh,
        out_type=[
            jax.ShapeDtypeStruct((n_rows, d01), jnp.float32),
            jax.ShapeDtypeStruct((n_rows, d2), jnp.float32),
        ],
        scratch_types=[
            pltpu.VMEM((nch, chunk), jnp.int32),
            pltpu.VMEM((chunk, d01), jnp.float32),
            pltpu.VMEM((chunk, d2), jnp.float32),
            pltpu.SemaphoreType.DMA,
        ],
    )
    def sc_gather(e01_hbm, oc_hbm, gidx_hbm, o01_hbm, o2_hbm,
                  idx_v, b01, b2, sem):
        wid = lax.axis_index("s") * nc + lax.axis_index("c")
        pltpu.sync_copy(gidx_hbm.at[pl.ds(wid * nch, nch)], idx_v)

        def body(j, carry):
            idx = idx_v.at[j]
            pltpu.async_copy(e01_hbm.at[idx], b01, sem).wait()
            pltpu.async_copy(oc_hbm.at[idx], b2, sem).wait()
            dst = wid * rpw + j * chunk
            pltpu.sync_copy(b01, o01_hbm.at[pl.ds(dst, chunk)])
            pltpu.sync_copy(b2, o2_hbm.at[pl.ds(dst, chunk)])
            return carry

        lax.fori_loop(0, nch, body, 0)

    return sc_gather, nch, chunk


def kernel(elem0, elem1, ochlv):
    b, t, l, f = ochlv.shape
    d0 = elem0.shape[-1]
    lf = l * f
    ocr = ochlv.reshape(b, t, lf)

    perf = _compute_perf(ocr)
    rank = _compute_rank(perf)
    orders, gidx = _compute_orders(rank)

    n_rows = b * t
    sc_gather, nch, chunk = _make_sc_gather(n_rows, 2 * d0, lf)
    gidx2 = gidx.reshape(n_rows // chunk, chunk)
    e01 = jnp.concatenate(
        [elem0.reshape(n_rows, d0), elem1.reshape(n_rows, d0)], axis=1)
    o01, o2 = sc_gather(e01, ocr.reshape(n_rows, lf), gidx2)
    return (
        o01[:, :d0].reshape(b, t, d0),
        o01[:, d0:].reshape(b, t, d0),
        o2.reshape(b, t, l, f),
        orders,
    )


# speedup vs baseline: 1.0627x; 1.0627x over previous
import functools

import jax
import jax.numpy as jnp
from jax import lax
from jax.experimental import pallas as pl
from jax.experimental.pallas import tpu as pltpu
from jax.experimental.pallas import tpu_sc as plsc

_CLOSE_IDX = 1


def _phys_view_ochlv(ochlv, l, f, b, t):
    z = ochlv.transpose(2, 3, 1, 0)
    z = z.reshape(l, f, t // 128, 128, b)
    z = z.transpose(0, 1, 2, 4, 3)
    return z.reshape(l * f * b * t)


def _unphys_ochlv(y1d, l, f, b, t):
    z = y1d.reshape(l, f, t // 128, b, 128)
    z = z.transpose(3, 2, 4, 0, 1)
    return z.reshape(b, t, l, f)


def _phys_view_elem(e, b, t, d):
    z = e.transpose(0, 2, 1)
    z = z.reshape(b, d // 8, 8, t // 128, 128)
    z = z.transpose(0, 1, 3, 2, 4)
    return z.reshape(b * d * t)


def _unphys_elem(o1d, b, t, d):
    z = o1d.reshape(b, d // 8, t // 128, 8, 128)
    z = z.transpose(0, 2, 4, 1, 3)
    return z.reshape(b, t, d)


def _perf_body(x_ref, out_ref, minl_ref, startv_ref, *, lb, nsteps):
    step = pl.program_id(0)
    x = x_ref[:, 0]
    lio = lax.broadcasted_iota(jnp.int32, (lb, 1, 1, 1), 0) + step * lb
    big = jnp.int32(10 * lb * nsteps)
    l_or_big = jnp.where(x != 0.0, lio, big)
    minl_blk = jnp.min(l_or_big, axis=0)
    startv_blk = jnp.sum(jnp.where(l_or_big == minl_blk[None], x, 0.0), axis=0)

    @pl.when(step == 0)
    def _init():
        minl_ref[...] = minl_blk
        startv_ref[...] = startv_blk

    @pl.when(step > 0)
    def _merge():
        upd = minl_blk < minl_ref[...]
        minl_ref[...] = jnp.where(upd, minl_blk, minl_ref[...])
        startv_ref[...] = jnp.where(upd, startv_blk, startv_ref[...])

    @pl.when(step == nsteps - 1)
    def _finish():
        last = x_ref[lb - 1, 0]
        start = startv_ref[...]
        safe = jnp.where(start != 0.0, start, 1.0)
        out_ref[...] = jnp.where(start != 0.0, (last - start) / safe, 0.0)


def _compute_perf(x5):
    l, f, nt, b, tm = x5.shape
    lb = 8
    nsteps = l // lb
    return pl.pallas_call(
        functools.partial(_perf_body, lb=lb, nsteps=nsteps),
        grid=(nsteps,),
        in_specs=[pl.BlockSpec((lb, 1, nt, b, tm),
                               lambda i: (i, _CLOSE_IDX, 0, 0, 0))],
        out_specs=pl.BlockSpec((nt, b, tm), lambda i: (0, 0, 0)),
        out_shape=jax.ShapeDtypeStruct((nt, b, tm), jnp.float32),
        scratch_shapes=[
            pltpu.VMEM((nt, b, tm), jnp.int32),
            pltpu.VMEM((nt, b, tm), jnp.float32),
        ],
    )(x5)


def _sort_body(p_ref, orders_ref, pio_ref, pie_ref, rank_ref, *, nt, b, tm):
    p = p_ref[...]
    pj = p[:, :, None, :]
    jio = (lax.broadcasted_iota(jnp.int32, (nt, 1, 1, tm), 0) * tm
           + lax.broadcasted_iota(jnp.int32, (nt, 1, 1, tm), 3))
    for ti in range(nt):
        pi = p[ti][None, :, :, None]
        iio = (lax.broadcasted_iota(jnp.int32, (1, 1, tm, 1), 2) + ti * tm)
        cmp = (pj > pi) | ((pj == pi) & (jio < iio))
        c32 = cmp.astype(jnp.int32)
        rank_ref[ti] = jnp.sum(jnp.sum(c32, axis=3), axis=0)
    r = rank_ref[...][:, :, None, :]
    bio = lax.broadcasted_iota(jnp.int32, (b, 1), 0)
    for tk in range(nt):
        kio = (lax.broadcasted_iota(jnp.int32, (1, 1, tm, 1), 2) + tk * tm)
        hit = (r == kio)
        o = jnp.sum(jnp.sum(jnp.where(hit, jio, 0), axis=3), axis=0)
        orders_ref[:, tk * tm:(tk + 1) * tm] = o
        hi = o >> 7
        lo = o & 127
        pio_ref[tk] = (hi << 9) + (bio << 7) + lo
        pie_ref[tk] = (hi << 10) + lo


def _compute_orders(perf_phys):
    nt, b, tm = perf_phys.shape
    t = nt * tm
    return pl.pallas_call(
        functools.partial(_sort_body, nt=nt, b=b, tm=tm),
        in_specs=[pl.BlockSpec((nt, b, tm), lambda: (0, 0, 0))],
        out_specs=[
            pl.BlockSpec((b, t), lambda: (0, 0)),
            pl.BlockSpec((nt, b, tm), lambda: (0, 0, 0)),
            pl.BlockSpec((nt, b, tm), lambda: (0, 0, 0)),
        ],
        out_shape=[
            jax.ShapeDtypeStruct((b, t), jnp.int32),
            jax.ShapeDtypeStruct((nt, b, tm), jnp.int32),
            jax.ShapeDtypeStruct((nt, b, tm), jnp.int32),
        ],
        scratch_shapes=[pltpu.VMEM((nt, b, tm), jnp.int32)],
    )(perf_phys)


def _make_sc_gather(b, t, n_planes, d):
    info = plsc.get_sparse_core_info()
    nc, ns = info.num_cores, info.num_subcores
    nw = nc * ns
    ppw = n_planes // nw
    pw = b * t
    ew = 8 * t
    n_oc = n_planes * pw
    n_e = b * d * t

    mesh = plsc.VectorSubcoreMesh(core_axis_name="c", subcore_axis_name="s")

    @functools.partial(
        pl.kernel,
        mesh=mesh,
        compiler_params=pltpu.CompilerParams(needs_layout_passes=False),
        out_type=[
            jax.ShapeDtypeStruct((n_oc,), jnp.float32),
            jax.ShapeDtypeStruct((n_e,), jnp.float32),
            jax.ShapeDtypeStruct((n_e,), jnp.float32),
        ],
        scratch_types=[
            pltpu.VMEM((b * t,), jnp.int32),
            pltpu.VMEM((b * t,), jnp.int32),
            pltpu.VMEM((pw,), jnp.float32),
            pltpu.VMEM((pw,), jnp.float32),
            pltpu.VMEM((ew,), jnp.float32),
            pltpu.VMEM((ew,), jnp.float32),
        ],
    )
    def sc_gather(x_hbm, e0_hbm, e1_hbm, pio_hbm, pie_hbm,
                  y_hbm, o0_hbm, o1_hbm,
                  piov, piev, pin, pout, ein, eout):
        wid = lax.axis_index("s") * nc + lax.axis_index("c")
        pltpu.sync_copy(pio_hbm, piov)
        pltpu.sync_copy(pie_hbm, piev)

        g0 = wid * ppw

        def plane_body(i, carry):
            base = (g0 + i) * pw
            pltpu.sync_copy(x_hbm.at[pl.ds(base, pw)], pin)

            def jbody(j, cc):
                dhi = (j >> 3) << 9
                dlo = (j & 7) << 4
                for bb in range(b):
                    ofs = dhi + (bb << 7) + dlo
                    pv = piov[pl.ds(ofs, 16)]
                    v = plsc.load_gather(pin, [pv])
                    pout[pl.ds(ofs, 16)] = v
                return cc

            lax.fori_loop(0, t // 16, jbody, 0, unroll=2)
            pltpu.sync_copy(pout, y_hbm.at[pl.ds(base, pw)])
            return carry

        lax.fori_loop(0, ppw, plane_body, 0)

        ebase = wid * ew
        bq = wid // 8

        def permute_egroup(e_hbm, o_hbm):
            pltpu.sync_copy(e_hbm.at[pl.ds(ebase, ew)], ein)

            def ejbody(j, cc):
                src_ofs = ((j >> 3) << 9) + (bq << 7) + ((j & 7) << 4)
                pv0 = piev[pl.ds(src_ofs, 16)]
                dhi = (j >> 3) << 10
                dlo = (j & 7) << 4
                for k in range(8):
                    v = plsc.load_gather(ein, [pv0 + k * 128])
                    eout[pl.ds(dhi + (k << 7) + dlo, 16)] = v
                return cc

            lax.fori_loop(0, t // 16, ejbody, 0, unroll=2)
            pltpu.sync_copy(eout, o_hbm.at[pl.ds(ebase, ew)])

        permute_egroup(e0_hbm, o0_hbm)
        permute_egroup(e1_hbm, o1_hbm)

    return sc_gather


def kernel(elem0, elem1, ochlv):
    b, t, l, f = ochlv.shape
    d = elem0.shape[-1]

    x1d = _phys_view_ochlv(ochlv, l, f, b, t)
    e0_1d = _phys_view_elem(elem0, b, t, d)
    e1_1d = _phys_view_elem(elem1, b, t, d)

    x5 = x1d.reshape(l, f, t // 128, b, 128)
    perf_phys = _compute_perf(x5)
    orders, pio, pie = _compute_orders(perf_phys)

    sc_gather = _make_sc_gather(b, t, l * f, d)
    y1d, o0_1d, o1_1d = sc_gather(
        x1d, e0_1d, e1_1d, pio.reshape(b * t), pie.reshape(b * t))

    o0 = _unphys_elem(o0_1d, b, t, d)
    o1 = _unphys_elem(o1_1d, b, t, d)
    o2 = _unphys_ochlv(y1d, l, f, b, t)
    return (o0, o1, o2, orders)


# speedup vs baseline: 1.5814x; 1.4881x over previous
import functools

import jax
import jax.numpy as jnp
from jax import lax
from jax.experimental import pallas as pl
from jax.experimental.pallas import tpu as pltpu
from jax.experimental.pallas import tpu_sc as plsc

_CLOSE_IDX = 1


def _phys_view_ochlv(ochlv, l, f, b, t):
    z = ochlv.transpose(2, 3, 1, 0)
    z = z.reshape(l, f, t // 128, 128, b)
    z = z.transpose(0, 1, 2, 4, 3)
    return z.reshape(l * f * b * t)


def _unphys_ochlv(y1d, l, f, b, t):
    z = y1d.reshape(l, f, t // 128, b, 128)
    z = z.transpose(3, 2, 4, 0, 1)
    return z.reshape(b, t, l, f)


def _phys_view_elem(e, b, t, d):
    z = e.transpose(0, 2, 1)
    z = z.reshape(b, d // 8, 8, t // 128, 128)
    z = z.transpose(0, 1, 3, 2, 4)
    return z.reshape(b * d * t)


def _unphys_elem(o1d, b, t, d):
    z = o1d.reshape(b, d // 8, t // 128, 8, 128)
    z = z.transpose(0, 2, 4, 1, 3)
    return z.reshape(b, t, d)


def _perf_body(x_ref, out_ref, minl_ref, startv_ref, *, lb, nsteps):
    step = pl.program_id(0)
    x = x_ref[:, 0]
    lio = lax.broadcasted_iota(jnp.int32, (lb, 1, 1, 1), 0) + step * lb
    big = jnp.int32(10 * lb * nsteps)
    l_or_big = jnp.where(x != 0.0, lio, big)
    minl_blk = jnp.min(l_or_big, axis=0)
    startv_blk = jnp.sum(jnp.where(l_or_big == minl_blk[None], x, 0.0), axis=0)

    @pl.when(step == 0)
    def _init():
        minl_ref[...] = minl_blk
        startv_ref[...] = startv_blk

    @pl.when(step > 0)
    def _merge():
        upd = minl_blk < minl_ref[...]
        minl_ref[...] = jnp.where(upd, minl_blk, minl_ref[...])
        startv_ref[...] = jnp.where(upd, startv_blk, startv_ref[...])

    @pl.when(step == nsteps - 1)
    def _finish():
        last = x_ref[lb - 1, 0]
        start = startv_ref[...]
        safe = jnp.where(start != 0.0, start, 1.0)
        out_ref[...] = jnp.where(start != 0.0, (last - start) / safe, 0.0)


def _compute_perf(x5):
    l, f, nt, b, tm = x5.shape
    lb = 8
    nsteps = l // lb
    return pl.pallas_call(
        functools.partial(_perf_body, lb=lb, nsteps=nsteps),
        grid=(nsteps,),
        in_specs=[pl.BlockSpec((lb, 1, nt, b, tm),
                               lambda i: (i, _CLOSE_IDX, 0, 0, 0))],
        out_specs=pl.BlockSpec((nt, b, tm), lambda i: (0, 0, 0)),
        out_shape=jax.ShapeDtypeStruct((nt, b, tm), jnp.float32),
        scratch_shapes=[
            pltpu.VMEM((nt, b, tm), jnp.int32),
            pltpu.VMEM((nt, b, tm), jnp.float32),
        ],
    )(x5)


def _sort_body(p_ref, orders_ref, pio_ref, pie_ref, rank_ref, *, nt, b, tm):
    p = p_ref[...]
    pj = p[:, :, None, :]
    jio = (lax.broadcasted_iota(jnp.int32, (nt, 1, 1, tm), 0) * tm
           + lax.broadcasted_iota(jnp.int32, (nt, 1, 1, tm), 3))
    for ti in range(nt):
        pi = p[ti][None, :, :, None]
        iio = (lax.broadcasted_iota(jnp.int32, (1, 1, tm, 1), 2) + ti * tm)
        cmp = (pj > pi) | ((pj == pi) & (jio < iio))
        c32 = cmp.astype(jnp.int32)
        rank_ref[ti] = jnp.sum(jnp.sum(c32, axis=3), axis=0)
    r = rank_ref[...][:, :, None, :]
    bio = lax.broadcasted_iota(jnp.int32, (b, 1), 0)
    for tk in range(nt):
        kio = (lax.broadcasted_iota(jnp.int32, (1, 1, tm, 1), 2) + tk * tm)
        hit = (r == kio)
        o = jnp.sum(jnp.sum(jnp.where(hit, jio, 0), axis=3), axis=0)
        orders_ref[:, tk * tm:(tk + 1) * tm] = o
        hi = o >> 7
        lo = o & 127
        pio_ref[tk] = (hi << 9) + (bio << 7) + lo
        pie_ref[tk] = (hi << 10) + lo


def _compute_orders(perf_phys):
    nt, b, tm = perf_phys.shape
    t = nt * tm
    return pl.pallas_call(
        functools.partial(_sort_body, nt=nt, b=b, tm=tm),
        in_specs=[pl.BlockSpec((nt, b, tm), lambda: (0, 0, 0))],
        out_specs=[
            pl.BlockSpec((b, t), lambda: (0, 0)),
            pl.BlockSpec((nt, b, tm), lambda: (0, 0, 0)),
            pl.BlockSpec((nt, b, tm), lambda: (0, 0, 0)),
        ],
        out_shape=[
            jax.ShapeDtypeStruct((b, t), jnp.int32),
            jax.ShapeDtypeStruct((nt, b, tm), jnp.int32),
            jax.ShapeDtypeStruct((nt, b, tm), jnp.int32),
        ],
        scratch_shapes=[pltpu.VMEM((nt, b, tm), jnp.int32)],
    )(perf_phys)


def _make_sc_gather(b, t, n_planes, d):
    info = plsc.get_sparse_core_info()
    nc, ns = info.num_cores, info.num_subcores
    nw = nc * ns
    ppw = n_planes // nw
    pw = b * t
    ew = 8 * t
    n_oc = n_planes * pw
    n_e = b * d * t

    mesh = plsc.VectorSubcoreMesh(core_axis_name="c", subcore_axis_name="s")

    @functools.partial(
        pl.kernel,
        mesh=mesh,
        compiler_params=pltpu.CompilerParams(needs_layout_passes=False),
        out_type=[
            jax.ShapeDtypeStruct((n_oc,), jnp.float32),
            jax.ShapeDtypeStruct((n_e,), jnp.float32),
            jax.ShapeDtypeStruct((n_e,), jnp.float32),
        ],
        scratch_types=[
            pltpu.VMEM((b * t,), jnp.int32),
            pltpu.VMEM((b * t,), jnp.int32),
            pltpu.VMEM((2 * pw,), jnp.float32),
            pltpu.VMEM((2 * pw,), jnp.float32),
            pltpu.VMEM((ew,), jnp.float32),
            pltpu.VMEM((ew,), jnp.float32),
            pltpu.SemaphoreType.DMA,
            pltpu.SemaphoreType.DMA,
            pltpu.SemaphoreType.DMA,
            pltpu.SemaphoreType.DMA,
        ],
    )
    def sc_gather(x_hbm, e0_hbm, e1_hbm, pio_hbm, pie_hbm,
                  y_hbm, o0_hbm, o1_hbm,
                  piov, piev, pin, pout, ein, eout,
                  sin0, sin1, sout0, sout1):
        wid = lax.axis_index("s") * nc + lax.axis_index("c")
        pltpu.sync_copy(pio_hbm, piov)
        pltpu.sync_copy(pie_hbm, piev)

        g0 = wid * ppw
        sins = (sin0, sin1)
        souts = (sout0, sout1)

        def permute_plane(src_ref, dst_ref):
            def jbody(jt, cc):
                jbase = jt << 9
                for bb in range(b):
                    for jm in range(t // (16 * 16)):
                        ofs = jbase + (bb << 7) + (jm << 4)
                        pv = piov[pl.ds(ofs, 16)]
                        dst_ref[pl.ds(ofs, 16)] = plsc.load_gather(
                            src_ref, [pv])
                return cc

            lax.fori_loop(0, t // 128, jbody, 0)

        pltpu.async_copy(x_hbm.at[pl.ds(g0 * pw, pw)], pin.at[pl.ds(0, pw)], sin0)

        def plane_body(i, carry):
            for par in range(2):
                p = g0 + 2 * i + par
                nxt = jnp.minimum(p + 1, jnp.int32(n_planes - 1))
                pltpu.make_async_copy(
                    x_hbm.at[pl.ds(p * pw, pw)], pin.at[pl.ds(par * pw, pw)], sins[par]).wait()
                pltpu.async_copy(
                    x_hbm.at[pl.ds(nxt * pw, pw)], pin.at[pl.ds((1 - par) * pw, pw)],
                    sins[1 - par])

                @pl.when(i > 0)
                def _drain():
                    pltpu.make_async_copy(
                        pout.at[pl.ds(par * pw, pw)], y_hbm.at[pl.ds(p * pw, pw)],
                        souts[par]).wait()

                permute_plane(pin.at[pl.ds(par * pw, pw)], pout.at[pl.ds(par * pw, pw)])
                pltpu.async_copy(
                    pout.at[pl.ds(par * pw, pw)], y_hbm.at[pl.ds(p * pw, pw)], souts[par])
            return carry

        lax.fori_loop(0, ppw // 2, plane_body, 0)
        pltpu.make_async_copy(
            pout.at[pl.ds(0, pw)], y_hbm.at[pl.ds(g0 * pw, pw)], sout0).wait()
        pltpu.make_async_copy(
            pout.at[pl.ds(pw, pw)], y_hbm.at[pl.ds(g0 * pw, pw)], sout1).wait()
        pltpu.make_async_copy(
            x_hbm.at[pl.ds(g0 * pw, pw)], pin.at[pl.ds(0, pw)], sin0).wait()

        ebase = wid * ew
        bq = wid // 8

        def permute_egroup(e_hbm, o_hbm, osem):
            pltpu.sync_copy(e_hbm.at[pl.ds(ebase, ew)], ein)

            def ejbody(jt, cc):
                sbase = (jt << 9) + (bq << 7)
                dbase = jt << 10
                for jm in range(t // (16 * 16)):
                    pv0 = piev[pl.ds(sbase + (jm << 4), 16)]
                    for k in range(8):
                        v = plsc.load_gather(ein, [pv0 + k * 128])
                        eout[pl.ds(dbase + (k << 7) + (jm << 4), 16)] = v
                return cc

            lax.fori_loop(0, t // 128, ejbody, 0)
            pltpu.async_copy(eout, o_hbm.at[pl.ds(ebase, ew)], osem)

        permute_egroup(e0_hbm, o0_hbm, sout0)
        pltpu.make_async_copy(eout, o0_hbm.at[pl.ds(ebase, ew)], sout0).wait()
        permute_egroup(e1_hbm, o1_hbm, sout1)
        pltpu.make_async_copy(eout, o1_hbm.at[pl.ds(ebase, ew)], sout1).wait()

    return sc_gather


def kernel(elem0, elem1, ochlv):
    b, t, l, f = ochlv.shape
    d = elem0.shape[-1]

    x1d = _phys_view_ochlv(ochlv, l, f, b, t)
    e0_1d = _phys_view_elem(elem0, b, t, d)
    e1_1d = _phys_view_elem(elem1, b, t, d)

    x5 = x1d.reshape(l, f, t // 128, b, 128)
    perf_phys = _compute_perf(x5)
    orders, pio, pie = _compute_orders(perf_phys)

    sc_gather = _make_sc_gather(b, t, l * f, d)
    y1d, o0_1d, o1_1d = sc_gather(
        x1d, e0_1d, e1_1d, pio.reshape(b * t), pie.reshape(b * t))

    o0 = _unphys_elem(o0_1d, b, t, d)
    o1 = _unphys_elem(o1_1d, b, t, d)
    o2 = _unphys_ochlv(y1d, l, f, b, t)
    return (o0, o1, o2, orders)
